# Initial kernel scaffold; baseline (speedup 1.0000x reference)
import functools

import jax
import jax.numpy as jnp
from jax import lax
from jax.experimental import pallas as pl
from jax.experimental.pallas import tpu as pltpu
from jax.experimental.pallas import tpu_sc as plsc

N = 10000
E = 320000
D = 128
NC, NS = 2, 16
NW = NC * NS
EPW = E // NW
CHUNK = 128
NFULL = EPW // CHUNK
TAIL = EPW - NFULL * CHUNK
RPT = N // NS


def _sc_body(with_count, *refs):
    if with_count:
        (x_h, src_h, dst_h, out_h, cnt_h,
         src_v, dst_v, rows_v, src_t, dst_t, rows_t,
         zbuf, zcnt, ones_v, ones_t, acc_sh, cnt_sh, sem) = refs
    else:
        (x_h, src_h, dst_h, out_h,
         src_v, dst_v, rows_v, src_t, dst_t, rows_t,
         zbuf, acc_sh, sem) = refs

    c = lax.axis_index("c")
    s = lax.axis_index("s")
    wid = s * NC + c

    def zrow(i, carry):
        for j in range(D // 16):
            zbuf[i, pl.ds(j * 16, 16)] = jnp.zeros((16,), jnp.float32)
        return carry
    lax.fori_loop(0, RPT, zrow, 0)
    pltpu.sync_copy(zbuf, acc_sh.at[pl.ds(s * RPT, RPT)])
    if with_count:
        def zc(i, carry):
            zcnt[i, :] = jnp.zeros((16,), jnp.float32)
            return carry
        lax.fori_loop(0, RPT, zc, 0)
        pltpu.sync_copy(zcnt, cnt_sh.at[pl.ds(s * RPT, RPT)])

        def onr(i, carry):
            ones_v[i, :] = jnp.ones((16,), jnp.float32)
            return carry
        lax.fori_loop(0, CHUNK, onr, 0)
        for i in range(TAIL):
            ones_t[i, :] = jnp.ones((16,), jnp.float32)
    plsc.subcore_barrier()

    base0 = wid * EPW

    def step(g, carry):
        base = base0 + g * CHUNK
        pltpu.sync_copy(src_h.at[pl.ds(base, CHUNK)], src_v)
        pltpu.sync_copy(dst_h.at[pl.ds(base, CHUNK)], dst_v)
        pltpu.async_copy(x_h.at[src_v], rows_v, sem).wait()
        pltpu.sync_copy(rows_v, acc_sh.at[dst_v], add=True)
        if with_count:
            pltpu.sync_copy(ones_v, cnt_sh.at[dst_v], add=True)
        return carry
    lax.fori_loop(0, NFULL, step, 0)

    base = base0 + NFULL * CHUNK
    pltpu.sync_copy(src_h.at[pl.ds(base, TAIL)], src_t)
    pltpu.sync_copy(dst_h.at[pl.ds(base, TAIL)], dst_t)
    pltpu.async_copy(x_h.at[src_t], rows_t, sem).wait()
    pltpu.sync_copy(rows_t, acc_sh.at[dst_t], add=True)
    if with_count:
        pltpu.sync_copy(ones_t, cnt_sh.at[dst_t], add=True)
    plsc.subcore_barrier()

    pltpu.sync_copy(acc_sh.at[pl.ds(s * RPT, RPT)],
                    out_h.at[c, pl.ds(s * RPT, RPT)])
    if with_count:
        pltpu.sync_copy(cnt_sh.at[pl.ds(s * RPT, RPT)],
                        cnt_h.at[c, pl.ds(s * RPT, RPT)])


def _make_sc_kernel(with_count):
    out_type = [jax.ShapeDtypeStruct((NC, N, D), jnp.float32)]
    scratch = [
        pltpu.VMEM((CHUNK,), jnp.int32),
        pltpu.VMEM((CHUNK,), jnp.int32),
        pltpu.VMEM((CHUNK, D), jnp.float32),
        pltpu.VMEM((TAIL,), jnp.int32),
        pltpu.VMEM((TAIL,), jnp.int32),
        pltpu.VMEM((TAIL, D), jnp.float32),
        pltpu.VMEM((RPT, D), jnp.float32),
    ]
    if with_count:
        out_type.append(jax.ShapeDtypeStruct((NC, N, 16), jnp.float32))
        scratch += [
            pltpu.VMEM((RPT, 16), jnp.float32),
            pltpu.VMEM((CHUNK, 16), jnp.float32),
            pltpu.VMEM((TAIL, 16), jnp.float32),
        ]
    scratch.append(pltpu.VMEM_SHARED((N, D), jnp.float32))
    if with_count:
        scratch.append(pltpu.VMEM_SHARED((N, 16), jnp.float32))
    scratch.append(pltpu.SemaphoreType.DMA)
    mesh = plsc.VectorSubcoreMesh(core_axis_name="c", subcore_axis_name="s")
    return pl.kernel(
        functools.partial(_sc_body, with_count),
        out_type=tuple(out_type),
        mesh=mesh,
        scratch_types=tuple(scratch),
    )


_sc_sum_count = _make_sc_kernel(True)
_sc_sum = _make_sc_kernel(False)

BLK = 400


def _tc_body(relu, sum_ref, cnt_ref, x_ref, wl_ref, wr_ref, b_ref, out_ref):
    ssum = sum_ref[0] + sum_ref[1]
    cnt = cnt_ref[0, :, 0:1] + cnt_ref[1, :, 0:1]
    aggr = ssum / jnp.maximum(cnt, 1.0)
    y = (jnp.dot(aggr, wl_ref[...], preferred_element_type=jnp.float32)
         + jnp.dot(x_ref[...], wr_ref[...], preferred_element_type=jnp.float32)
         + b_ref[...])
    out_ref[...] = jnp.maximum(y, 0.0) if relu else y


def _tc_layer(summed, cnt, x, Wl, Wr, b, relu):
    grid = (N // BLK,)
    return pl.pallas_call(
        functools.partial(_tc_body, relu),
        grid=grid,
        in_specs=[
            pl.BlockSpec((NC, BLK, D), lambda i: (0, i, 0)),
            pl.BlockSpec((NC, BLK, 16), lambda i: (0, i, 0)),
            pl.BlockSpec((BLK, D), lambda i: (i, 0)),
            pl.BlockSpec((D, D), lambda i: (0, 0)),
            pl.BlockSpec((D, D), lambda i: (0, 0)),
            pl.BlockSpec((1, D), lambda i: (0, 0)),
        ],
        out_specs=pl.BlockSpec((BLK, D), lambda i: (i, 0)),
        out_shape=jax.ShapeDtypeStruct((N, D), jnp.float32),
    )(summed, cnt, x, Wl, Wr, b.reshape(1, D))


def kernel(x, edge_index, W1l, W1r, b1, W2l, W2r, b2):
    src = edge_index[0].astype(jnp.int32)
    dst = edge_index[1].astype(jnp.int32)
    summed1, cnt = _sc_sum_count(x, src, dst)
    h = _tc_layer(summed1, cnt, x, W1l, W1r, b1, relu=True)
    (summed2,) = _sc_sum(h, src, dst)
    return _tc_layer(summed2, cnt, h, W2l, W2r, b2, relu=False)


# speedup vs baseline: 5.8277x; 5.8277x over previous
import functools

import jax
import jax.numpy as jnp
from jax import lax
from jax.experimental import pallas as pl
from jax.experimental.pallas import tpu as pltpu
from jax.experimental.pallas import tpu_sc as plsc

N = 10000
E = 320000
D = 128
NC, NS = 2, 16
NW = NC * NS
EPW = E // NW
CHUNK = 128
NFULL = EPW // CHUNK
TAIL = EPW - NFULL * CHUNK
RPT = 624
RPT_LAST = N - RPT * (NS - 1)


def _zero_stripe(zbuf, dst_sh, sbase, s):
    def zcopy(k, carry):
        pltpu.sync_copy(zbuf, dst_sh.at[pl.ds(sbase + k * 16, 16)])
        return carry
    lax.fori_loop(0, RPT // 16, zcopy, 0)
    pl.when(s == NS - 1)(
        lambda: pltpu.sync_copy(zbuf, dst_sh.at[pl.ds(sbase + RPT, 16)]))


def _fill(buf, rows, value):
    def frow(i, carry):
        for j in range(D // 16):
            buf[i, pl.ds(j * 16, 16)] = jnp.full((16,), value, jnp.float32)
        return carry
    lax.fori_loop(0, rows, frow, 0)


def _wb(src_sh, out_h, c, sbase, s):
    def wb(sz):
        pltpu.sync_copy(src_sh.at[pl.ds(sbase, sz)],
                        out_h.at[c, pl.ds(sbase, sz)])
    pl.when(s < NS - 1)(lambda: wb(RPT))
    pl.when(s == NS - 1)(lambda: wb(RPT_LAST))


def _sc_sum_body(x_h, src_h, dst_h, out_h,
                 src_v, dst_v, rows_v, src_t, dst_t, zbuf, acc_sh, sem):
    c = lax.axis_index("c")
    s = lax.axis_index("s")
    wid = s * NC + c
    sbase = s * RPT

    _fill(zbuf, 16, 0.0)
    _zero_stripe(zbuf, acc_sh, sbase, s)
    plsc.subcore_barrier()

    base0 = wid * EPW

    def step(g, carry):
        base = base0 + g * CHUNK
        pltpu.sync_copy(src_h.at[pl.ds(base, CHUNK)], src_v)
        pltpu.sync_copy(dst_h.at[pl.ds(base, CHUNK)], dst_v)
        pltpu.async_copy(x_h.at[src_v], rows_v, sem).wait()
        pltpu.sync_copy(rows_v, acc_sh.at[dst_v], add=True)
        return carry
    lax.fori_loop(0, NFULL, step, 0)

    base = base0 + NFULL * CHUNK
    pltpu.sync_copy(src_h.at[pl.ds(base, TAIL)], src_t)
    pltpu.sync_copy(dst_h.at[pl.ds(base, TAIL)], dst_t)
    pltpu.async_copy(x_h.at[src_t], rows_v.at[pl.ds(0, TAIL)], sem).wait()
    pltpu.sync_copy(rows_v.at[pl.ds(0, TAIL)], acc_sh.at[dst_t], add=True)
    plsc.subcore_barrier()

    _wb(acc_sh, out_h, c, sbase, s)


def _sc_cnt_body(dst_h, out_h, dst_v, dst_t, ones_v, zbuf, cnt_sh):
    c = lax.axis_index("c")
    s = lax.axis_index("s")
    wid = s * NC + c
    sbase = s * RPT

    _fill(zbuf, 16, 0.0)
    _fill(ones_v, CHUNK, 1.0)
    _zero_stripe(zbuf, cnt_sh, sbase, s)
    plsc.subcore_barrier()

    base0 = wid * EPW

    def step(g, carry):
        base = base0 + g * CHUNK
        pltpu.sync_copy(dst_h.at[pl.ds(base, CHUNK)], dst_v)
        pltpu.sync_copy(ones_v, cnt_sh.at[dst_v], add=True)
        return carry
    lax.fori_loop(0, NFULL, step, 0)

    base = base0 + NFULL * CHUNK
    pltpu.sync_copy(dst_h.at[pl.ds(base, TAIL)], dst_t)
    pltpu.sync_copy(ones_v.at[pl.ds(0, TAIL)], cnt_sh.at[dst_t], add=True)
    plsc.subcore_barrier()

    _wb(cnt_sh, out_h, c, sbase, s)


_MESH = plsc.VectorSubcoreMesh(core_axis_name="c", subcore_axis_name="s")

_sc_sum = pl.kernel(
    _sc_sum_body,
    out_type=(jax.ShapeDtypeStruct((NC, N, D), jnp.float32),),
    mesh=_MESH,
    scratch_types=(
        pltpu.VMEM((CHUNK,), jnp.int32),
        pltpu.VMEM((CHUNK,), jnp.int32),
        pltpu.VMEM((CHUNK, D), jnp.float32),
        pltpu.VMEM((TAIL,), jnp.int32),
        pltpu.VMEM((TAIL,), jnp.int32),
        pltpu.VMEM((16, D), jnp.float32),
        pltpu.VMEM_SHARED((N, D), jnp.float32),
        pltpu.SemaphoreType.DMA,
    ),
)

_sc_cnt = pl.kernel(
    _sc_cnt_body,
    out_type=(jax.ShapeDtypeStruct((NC, N, D), jnp.float32),),
    mesh=_MESH,
    scratch_types=(
        pltpu.VMEM((CHUNK,), jnp.int32),
        pltpu.VMEM((TAIL,), jnp.int32),
        pltpu.VMEM((CHUNK, D), jnp.float32),
        pltpu.VMEM((16, D), jnp.float32),
        pltpu.VMEM_SHARED((N, D), jnp.float32),
    ),
)

BLK = 400


def _tc_body(relu, sum_ref, cnt_ref, x_ref, wl_ref, wr_ref, b_ref, out_ref):
    ssum = sum_ref[0] + sum_ref[1]
    cnt = cnt_ref[0, :, 0:1] + cnt_ref[1, :, 0:1]
    aggr = ssum / jnp.maximum(cnt, 1.0)
    y = (jnp.dot(aggr, wl_ref[...], preferred_element_type=jnp.float32)
         + jnp.dot(x_ref[...], wr_ref[...], preferred_element_type=jnp.float32)
         + b_ref[...])
    out_ref[...] = jnp.maximum(y, 0.0) if relu else y


def _tc_layer(summed, cnt, x, Wl, Wr, b, relu):
    grid = (N // BLK,)
    return pl.pallas_call(
        functools.partial(_tc_body, relu),
        grid=grid,
        in_specs=[
            pl.BlockSpec((NC, BLK, D), lambda i: (0, i, 0)),
            pl.BlockSpec((NC, BLK, D), lambda i: (0, i, 0)),
            pl.BlockSpec((BLK, D), lambda i: (i, 0)),
            pl.BlockSpec((D, D), lambda i: (0, 0)),
            pl.BlockSpec((D, D), lambda i: (0, 0)),
            pl.BlockSpec((1, D), lambda i: (0, 0)),
        ],
        out_specs=pl.BlockSpec((BLK, D), lambda i: (i, 0)),
        out_shape=jax.ShapeDtypeStruct((N, D), jnp.float32),
    )(summed, cnt, x, Wl, Wr, b.reshape(1, D))


def kernel(x, edge_index, W1l, W1r, b1, W2l, W2r, b2):
    src = edge_index[0].astype(jnp.int32)
    dst = edge_index[1].astype(jnp.int32)
    (cnt,) = _sc_cnt(dst)
    (summed1,) = _sc_sum(x, src, dst)
    h = _tc_layer(summed1, cnt, x, W1l, W1r, b1, relu=True)
    (summed2,) = _sc_sum(h, src, dst)
    return _tc_layer(summed2, cnt, h, W2l, W2r, b2, relu=False)


# speedup vs baseline: 8.1626x; 1.4006x over previous
import functools

import jax
import jax.numpy as jnp
from jax import lax
from jax.experimental import pallas as pl
from jax.experimental.pallas import tpu as pltpu
from jax.experimental.pallas import tpu_sc as plsc

N = 10000
E = 320000
D = 128
NC, NS = 2, 16
NW = NC * NS
EPW = E // NW
CHUNK = 128
NFULL = EPW // CHUNK
TAIL = EPW - NFULL * CHUNK
RPT = 624
RPT_LAST = N - RPT * (NS - 1)


def _zero_stripe(zbuf, dst_sh, sbase, s):
    def zcopy(k, carry):
        pltpu.sync_copy(zbuf, dst_sh.at[pl.ds(sbase + k * 16, 16)])
        return carry
    lax.fori_loop(0, RPT // 16, zcopy, 0)
    pl.when(s == NS - 1)(
        lambda: pltpu.sync_copy(zbuf, dst_sh.at[pl.ds(sbase + RPT, 16)]))


def _fill(buf, rows, value):
    def frow(i, carry):
        for j in range(D // 16):
            buf[i, pl.ds(j * 16, 16)] = jnp.full((16,), value, jnp.float32)
        return carry
    lax.fori_loop(0, rows, frow, 0)


def _wb(src_sh, out_h, c, sbase, s):
    def wb(sz):
        pltpu.sync_copy(src_sh.at[pl.ds(sbase, sz)],
                        out_h.at[c, pl.ds(sbase, sz)])
    pl.when(s < NS - 1)(lambda: wb(RPT))
    pl.when(s == NS - 1)(lambda: wb(RPT_LAST))


def _sc_sum_body(x_h, src_h, dst_h, out_h,
                 src_v0, dst_v0, rows_v0, src_v1, dst_v1, rows_v1,
                 src_t, dst_t, zbuf, acc_sh, sem0, sem1):
    c = lax.axis_index("c")
    s = lax.axis_index("s")
    wid = s * NC + c
    sbase = s * RPT

    _fill(zbuf, 16, 0.0)
    _zero_stripe(zbuf, acc_sh, sbase, s)
    plsc.subcore_barrier()

    base0 = wid * EPW
    bufs = ((src_v0, dst_v0, rows_v0, sem0), (src_v1, dst_v1, rows_v1, sem1))

    def load_and_fire(g, b):
        src_v, dst_v, rows_v, sem = bufs[b]
        base = base0 + g * CHUNK
        pltpu.sync_copy(src_h.at[pl.ds(base, CHUNK)], src_v)
        pltpu.sync_copy(dst_h.at[pl.ds(base, CHUNK)], dst_v)
        pltpu.make_async_copy(x_h.at[src_v], rows_v, sem).start()

    load_and_fire(0, 0)
    load_and_fire(1, 1)

    def step2(k, carry):
        for b in range(2):
            g = 2 * k + b
            src_v, dst_v, rows_v, sem = bufs[b]
            pltpu.make_async_copy(x_h.at[src_v], rows_v, sem).wait()
            pltpu.sync_copy(rows_v, acc_sh.at[dst_v], add=True)

            @pl.when(g + 2 < NFULL)
            def _():
                load_and_fire(g + 2, b)
        return carry
    lax.fori_loop(0, NFULL // 2, step2, 0)

    base = base0 + NFULL * CHUNK
    pltpu.sync_copy(src_h.at[pl.ds(base, TAIL)], src_t)
    pltpu.sync_copy(dst_h.at[pl.ds(base, TAIL)], dst_t)
    pltpu.async_copy(x_h.at[src_t], rows_v0.at[pl.ds(0, TAIL)], sem0).wait()
    pltpu.sync_copy(rows_v0.at[pl.ds(0, TAIL)], acc_sh.at[dst_t], add=True)
    plsc.subcore_barrier()

    _wb(acc_sh, out_h, c, sbase, s)


def _sc_cnt_body(dst_h, out_h, dst_v, dst_t, ones_v, zbuf, cnt_sh):
    c = lax.axis_index("c")
    s = lax.axis_index("s")
    wid = s * NC + c
    sbase = s * RPT

    _fill(zbuf, 16, 0.0)
    _fill(ones_v, CHUNK, 1.0)
    _zero_stripe(zbuf, cnt_sh, sbase, s)
    plsc.subcore_barrier()

    base0 = wid * EPW

    def step(g, carry):
        base = base0 + g * CHUNK
        pltpu.sync_copy(dst_h.at[pl.ds(base, CHUNK)], dst_v)
        pltpu.sync_copy(ones_v, cnt_sh.at[dst_v], add=True)
        return carry
    lax.fori_loop(0, NFULL, step, 0)

    base = base0 + NFULL * CHUNK
    pltpu.sync_copy(dst_h.at[pl.ds(base, TAIL)], dst_t)
    pltpu.sync_copy(ones_v.at[pl.ds(0, TAIL)], cnt_sh.at[dst_t], add=True)
    plsc.subcore_barrier()

    _wb(cnt_sh, out_h, c, sbase, s)


_MESH = plsc.VectorSubcoreMesh(core_axis_name="c", subcore_axis_name="s")

_sc_sum = pl.kernel(
    _sc_sum_body,
    out_type=(jax.ShapeDtypeStruct((NC, N, D), jnp.float32),),
    mesh=_MESH,
    scratch_types=(
        pltpu.VMEM((CHUNK,), jnp.int32),
        pltpu.VMEM((CHUNK,), jnp.int32),
        pltpu.VMEM((CHUNK, D), jnp.float32),
        pltpu.VMEM((CHUNK,), jnp.int32),
        pltpu.VMEM((CHUNK,), jnp.int32),
        pltpu.VMEM((CHUNK, D), jnp.float32),
        pltpu.VMEM((TAIL,), jnp.int32),
        pltpu.VMEM((TAIL,), jnp.int32),
        pltpu.VMEM((16, D), jnp.float32),
        pltpu.VMEM_SHARED((N, D), jnp.float32),
        pltpu.SemaphoreType.DMA,
        pltpu.SemaphoreType.DMA,
    ),
)

_sc_cnt = pl.kernel(
    _sc_cnt_body,
    out_type=(jax.ShapeDtypeStruct((NC, N, D), jnp.float32),),
    mesh=_MESH,
    scratch_types=(
        pltpu.VMEM((CHUNK,), jnp.int32),
        pltpu.VMEM((TAIL,), jnp.int32),
        pltpu.VMEM((CHUNK, D), jnp.float32),
        pltpu.VMEM((16, D), jnp.float32),
        pltpu.VMEM_SHARED((N, D), jnp.float32),
    ),
)

BLK = 400


def _tc_body(relu, sum_ref, cnt_ref, x_ref, wl_ref, wr_ref, b_ref, out_ref):
    ssum = sum_ref[0] + sum_ref[1]
    cnt = cnt_ref[0, :, 0:1] + cnt_ref[1, :, 0:1]
    aggr = ssum / jnp.maximum(cnt, 1.0)
    y = (jnp.dot(aggr, wl_ref[...], preferred_element_type=jnp.float32)
         + jnp.dot(x_ref[...], wr_ref[...], preferred_element_type=jnp.float32)
         + b_ref[...])
    out_ref[...] = jnp.maximum(y, 0.0) if relu else y


def _tc_layer(summed, cnt, x, Wl, Wr, b, relu):
    grid = (N // BLK,)
    return pl.pallas_call(
        functools.partial(_tc_body, relu),
        grid=grid,
        in_specs=[
            pl.BlockSpec((NC, BLK, D), lambda i: (0, i, 0)),
            pl.BlockSpec((NC, BLK, D), lambda i: (0, i, 0)),
            pl.BlockSpec((BLK, D), lambda i: (i, 0)),
            pl.BlockSpec((D, D), lambda i: (0, 0)),
            pl.BlockSpec((D, D), lambda i: (0, 0)),
            pl.BlockSpec((1, D), lambda i: (0, 0)),
        ],
        out_specs=pl.BlockSpec((BLK, D), lambda i: (i, 0)),
        out_shape=jax.ShapeDtypeStruct((N, D), jnp.float32),
    )(summed, cnt, x, Wl, Wr, b.reshape(1, D))


def kernel(x, edge_index, W1l, W1r, b1, W2l, W2r, b2):
    src = edge_index[0].astype(jnp.int32)
    dst = edge_index[1].astype(jnp.int32)
    (cnt,) = _sc_cnt(dst)
    (summed1,) = _sc_sum(x, src, dst)
    h = _tc_layer(summed1, cnt, x, W1l, W1r, b1, relu=True)
    (summed2,) = _sc_sum(h, src, dst)
    return _tc_layer(summed2, cnt, h, W2l, W2r, b2, relu=False)


# speedup vs baseline: 9.8614x; 1.2081x over previous
import functools

import jax
import jax.numpy as jnp
from jax import lax
from jax.experimental import pallas as pl
from jax.experimental.pallas import tpu as pltpu
from jax.experimental.pallas import tpu_sc as plsc

N = 10000
E = 320000
D = 128
CW = 128
NC, NS = 2, 16
NW = NC * NS
EPW = E // NW
C = 80
NCH = EPW // C
RPT = 624
RPT_LAST = N - RPT * (NS - 1)


def _zero_stripe(zbuf, dst_sh, sbase, s):
    def zcopy(k, carry):
        pltpu.sync_copy(zbuf, dst_sh.at[pl.ds(sbase + k * 8, 8)])
        return carry
    lax.fori_loop(0, RPT // 8, zcopy, 0)

    def ztail(k, carry):
        pltpu.sync_copy(zbuf, dst_sh.at[pl.ds(sbase + RPT + k * 8, 8)])
        return carry

    @pl.when(s == NS - 1)
    def _():
        lax.fori_loop(0, (RPT_LAST - RPT) // 8, ztail, 0)


def _fill(buf, rows, cols, value):
    def frow(i, carry):
        for j in range(cols // 16):
            buf[i, pl.ds(j * 16, 16)] = jnp.full((16,), value, jnp.float32)
        return carry
    lax.fori_loop(0, rows, frow, 0)


def _wb(src_sh, out_h, c, sbase, s):
    def wb(sz):
        pltpu.sync_copy(src_sh.at[pl.ds(sbase, sz)],
                        out_h.at[c, pl.ds(sbase, sz)])
    pl.when(s < NS - 1)(lambda: wb(RPT))
    pl.when(s == NS - 1)(lambda: wb(RPT_LAST))


def _sc_sum_body(x_h, src_h, dst_h, out_h,
                 src_all, dst_all, rows_v0, rows_v1, zbuf, acc_sh,
                 sem0, sem1):
    c = lax.axis_index("c")
    s = lax.axis_index("s")
    wid = s * NC + c
    sbase = s * RPT

    _fill(zbuf, 8, D, 0.0)
    _zero_stripe(zbuf, acc_sh, sbase, s)
    pltpu.sync_copy(src_h.at[wid], src_all)
    pltpu.sync_copy(dst_h.at[wid], dst_all)
    plsc.subcore_barrier()

    bufs = ((rows_v0, sem0), (rows_v1, sem1))

    def fire(g, b):
        rows_v, sem = bufs[b]
        idx = src_all.at[pl.ds(g * C, C)]
        pltpu.make_async_copy(x_h.at[idx], rows_v, sem).start()

    def finish(g, b):
        rows_v, sem = bufs[b]
        idx = src_all.at[pl.ds(g * C, C)]
        pltpu.make_async_copy(x_h.at[idx], rows_v, sem).wait()
        pltpu.sync_copy(rows_v, acc_sh.at[dst_all.at[g]], add=True)

    fire(0, 0)
    fire(1, 1)

    def step2(k, carry):
        for b in range(2):
            g = 2 * k + b
            finish(g, b)

            @pl.when(g + 2 < NCH)
            def _():
                fire(g + 2, b)
        return carry
    lax.fori_loop(0, NCH // 2, step2, 0)
    if NCH % 2:
        finish(NCH - 1, 0)
    plsc.subcore_barrier()

    _wb(acc_sh, out_h, c, sbase, s)


def _sc_cnt_body(dst_h, out_h, dst_all, ones_v, zbuf, cnt_sh):
    c = lax.axis_index("c")
    s = lax.axis_index("s")
    wid = s * NC + c
    sbase = s * RPT

    _fill(zbuf, 8, CW, 0.0)
    _fill(ones_v, C, CW, 1.0)

    def zcopy(k, carry):
        pltpu.sync_copy(zbuf, cnt_sh.at[pl.ds(sbase + k * 8, 8)])
        return carry
    lax.fori_loop(0, RPT // 8, zcopy, 0)

    def ztail(k, carry):
        pltpu.sync_copy(zbuf, cnt_sh.at[pl.ds(sbase + RPT + k * 8, 8)])
        return carry

    @pl.when(s == NS - 1)
    def _():
        lax.fori_loop(0, (RPT_LAST - RPT) // 8, ztail, 0)
    pltpu.sync_copy(dst_h.at[wid], dst_all)
    plsc.subcore_barrier()

    def step(g, carry):
        pltpu.sync_copy(ones_v, cnt_sh.at[dst_all.at[g]], add=True)
        return carry
    lax.fori_loop(0, NCH, step, 0)
    plsc.subcore_barrier()

    def wb(sz):
        pltpu.sync_copy(cnt_sh.at[pl.ds(sbase, sz)],
                        out_h.at[c, pl.ds(sbase, sz)])
    pl.when(s < NS - 1)(lambda: wb(RPT))
    pl.when(s == NS - 1)(lambda: wb(RPT_LAST))


_MESH = plsc.VectorSubcoreMesh(core_axis_name="c", subcore_axis_name="s")

_sc_sum = pl.kernel(
    _sc_sum_body,
    out_type=(jax.ShapeDtypeStruct((NC, N, D), jnp.float32),),
    mesh=_MESH,
    scratch_types=(
        pltpu.VMEM((EPW,), jnp.int32),
        pltpu.VMEM((NCH, C), jnp.int32),
        pltpu.VMEM((C, D), jnp.float32),
        pltpu.VMEM((C, D), jnp.float32),
        pltpu.VMEM((8, D), jnp.float32),
        pltpu.VMEM_SHARED((N, D), jnp.float32),
        pltpu.SemaphoreType.DMA,
        pltpu.SemaphoreType.DMA,
    ),
)

_sc_cnt = pl.kernel(
    _sc_cnt_body,
    out_type=(jax.ShapeDtypeStruct((NC, N, CW), jnp.float32),),
    mesh=_MESH,
    scratch_types=(
        pltpu.VMEM((NCH, C), jnp.int32),
        pltpu.VMEM((C, CW), jnp.float32),
        pltpu.VMEM((8, CW), jnp.float32),
        pltpu.VMEM_SHARED((N, CW), jnp.float32),
    ),
)

BLK = 400


def _tc_body(relu, sum_ref, cnt_ref, x_ref, wl_ref, wr_ref, b_ref, out_ref):
    ssum = sum_ref[0] + sum_ref[1]
    cnt = cnt_ref[0, :, 0:1] + cnt_ref[1, :, 0:1]
    aggr = ssum / jnp.maximum(cnt, 1.0)
    y = (jnp.dot(aggr, wl_ref[...], preferred_element_type=jnp.float32)
         + jnp.dot(x_ref[...], wr_ref[...], preferred_element_type=jnp.float32)
         + b_ref[...])
    out_ref[...] = jnp.maximum(y, 0.0) if relu else y


def _tc_layer(summed, cnt, x, Wl, Wr, b, relu):
    grid = (N // BLK,)
    return pl.pallas_call(
        functools.partial(_tc_body, relu),
        grid=grid,
        in_specs=[
            pl.BlockSpec((NC, BLK, D), lambda i: (0, i, 0)),
            pl.BlockSpec((NC, BLK, CW), lambda i: (0, i, 0)),
            pl.BlockSpec((BLK, D), lambda i: (i, 0)),
            pl.BlockSpec((D, D), lambda i: (0, 0)),
            pl.BlockSpec((D, D), lambda i: (0, 0)),
            pl.BlockSpec((1, D), lambda i: (0, 0)),
        ],
        out_specs=pl.BlockSpec((BLK, D), lambda i: (i, 0)),
        out_shape=jax.ShapeDtypeStruct((N, D), jnp.float32),
    )(summed, cnt, x, Wl, Wr, b.reshape(1, D))


def kernel(x, edge_index, W1l, W1r, b1, W2l, W2r, b2):
    src = edge_index[0].astype(jnp.int32).reshape(NW, EPW)
    dst = edge_index[1].astype(jnp.int32).reshape(NW, NCH, C)
    (cnt,) = _sc_cnt(dst)
    (summed1,) = _sc_sum(x, src, dst)
    h = _tc_layer(summed1, cnt, x, W1l, W1r, b1, relu=True)
    (summed2,) = _sc_sum(h, src, dst)
    return _tc_layer(summed2, cnt, h, W2l, W2r, b2, relu=False)


# speedup vs baseline: 11.6721x; 1.1836x over previous
import functools

import jax
import jax.numpy as jnp
from jax import lax
from jax.experimental import pallas as pl
from jax.experimental.pallas import tpu as pltpu
from jax.experimental.pallas import tpu_sc as plsc

N = 10000
E = 320000
D = 128
CW = 128
NC, NS = 2, 16
NW = NC * NS
EPW = E // NW
C = 80
NCH = EPW // C
RPT = 624
RPT_LAST = N - RPT * (NS - 1)


def _zero_stripe(zbuf, dst_sh, sbase, s):
    def zcopy(k, carry):
        pltpu.sync_copy(zbuf, dst_sh.at[pl.ds(sbase + k * 8, 8)])
        return carry
    lax.fori_loop(0, RPT // 8, zcopy, 0)

    def ztail(k, carry):
        pltpu.sync_copy(zbuf, dst_sh.at[pl.ds(sbase + RPT + k * 8, 8)])
        return carry

    @pl.when(s == NS - 1)
    def _():
        lax.fori_loop(0, (RPT_LAST - RPT) // 8, ztail, 0)


def _fill(buf, rows, cols, value):
    def frow(i, carry):
        for j in range(cols // 16):
            buf[i, pl.ds(j * 16, 16)] = jnp.full((16,), value, jnp.float32)
        return carry
    lax.fori_loop(0, rows, frow, 0)


def _wb(src_sh, out_h, c, sbase, s):
    def wb(sz):
        pltpu.sync_copy(src_sh.at[pl.ds(sbase, sz)],
                        out_h.at[c, pl.ds(sbase, sz)])
    pl.when(s < NS - 1)(lambda: wb(RPT))
    pl.when(s == NS - 1)(lambda: wb(RPT_LAST))


def _sc_sum_body(x_h, src_h, dst_h, out_h,
                 src_all, dst_all, rows_v0, rows_v1, zbuf, acc_sh,
                 sem0, sem1):
    c = lax.axis_index("c")
    s = lax.axis_index("s")
    wid = s * NC + c
    sbase = s * RPT

    _fill(zbuf, 8, D, 0.0)
    _zero_stripe(zbuf, acc_sh, sbase, s)
    pltpu.sync_copy(src_h.at[wid], src_all)
    pltpu.sync_copy(dst_h.at[wid], dst_all)
    plsc.subcore_barrier()

    bufs = ((rows_v0, sem0), (rows_v1, sem1))

    def fire(g, b):
        rows_v, sem = bufs[b]
        idx = src_all.at[pl.ds(g * C, C)]
        pltpu.make_async_copy(x_h.at[idx], rows_v, sem).start()

    def finish(g, b):
        rows_v, sem = bufs[b]
        idx = src_all.at[pl.ds(g * C, C)]
        pltpu.make_async_copy(x_h.at[idx], rows_v, sem).wait()
        pltpu.sync_copy(rows_v, acc_sh.at[dst_all.at[g]], add=True)

    fire(0, 0)
    fire(1, 1)

    def step2(k, carry):
        for b in range(2):
            g = 2 * k + b
            finish(g, b)

            @pl.when(g + 2 < NCH)
            def _():
                fire(g + 2, b)
        return carry
    lax.fori_loop(0, NCH // 2, step2, 0)
    if NCH % 2:
        finish(NCH - 1, 0)
    plsc.subcore_barrier()

    _wb(acc_sh, out_h, c, sbase, s)


NR = 80


def _sc_cnt_body(dst_h, zeros_h, out_h, dst_flat, hist2d, iota_v, cnt_sh):
    c = lax.axis_index("c")
    s = lax.axis_index("s")
    wid = s * NC + c

    for j in range(NR // 16):
        iota_v[pl.ds(j * 16, 16)] = lax.iota(jnp.int32, 16) + 16 * j
    pltpu.sync_copy(zeros_h, hist2d)

    @pl.when(s == 0)
    def _():
        pltpu.sync_copy(zeros_h, cnt_sh)
    pltpu.sync_copy(dst_h.at[wid], dst_flat)
    plsc.subcore_barrier()

    ones16 = jnp.ones((16,), jnp.float32)

    def step(i, carry):
        idx = dst_flat[pl.ds(i * 16, 16)]
        r = lax.shift_right_logical(idx, 7)
        cl = lax.bitwise_and(idx, 127)
        plsc.addupdate_scatter(hist2d, [r, cl], ones16)
        return carry
    lax.fori_loop(0, EPW // 16, step, 0)

    pltpu.sync_copy(hist2d, cnt_sh.at[iota_v], add=True)
    plsc.subcore_barrier()

    @pl.when(s < NR // 8)
    def _():
        pltpu.sync_copy(cnt_sh.at[pl.ds(s * 8, 8)],
                        out_h.at[c, pl.ds(s * 8, 8)])


_MESH = plsc.VectorSubcoreMesh(core_axis_name="c", subcore_axis_name="s")

_sc_sum = pl.kernel(
    _sc_sum_body,
    out_type=(jax.ShapeDtypeStruct((NC, N, D), jnp.float32),),
    mesh=_MESH,
    scratch_types=(
        pltpu.VMEM((EPW,), jnp.int32),
        pltpu.VMEM((NCH, C), jnp.int32),
        pltpu.VMEM((C, D), jnp.float32),
        pltpu.VMEM((C, D), jnp.float32),
        pltpu.VMEM((8, D), jnp.float32),
        pltpu.VMEM_SHARED((N, D), jnp.float32),
        pltpu.SemaphoreType.DMA,
        pltpu.SemaphoreType.DMA,
    ),
)

_sc_cnt = pl.kernel(
    _sc_cnt_body,
    out_type=(jax.ShapeDtypeStruct((NC, NR, D), jnp.float32),),
    mesh=_MESH,
    scratch_types=(
        pltpu.VMEM((EPW,), jnp.int32),
        pltpu.VMEM((NR, D), jnp.float32),
        pltpu.VMEM((NR,), jnp.int32),
        pltpu.VMEM_SHARED((NR, D), jnp.float32),
    ),
    compiler_params=pltpu.CompilerParams(needs_layout_passes=False),
)

BLK = 400


def _tc_body(relu, sum_ref, cnt_ref, x_ref, wl_ref, wr_ref, b_ref, out_ref):
    ssum = sum_ref[0] + sum_ref[1]
    cnt = cnt_ref[0, :, 0:1] + cnt_ref[1, :, 0:1]
    aggr = ssum / jnp.maximum(cnt, 1.0)
    y = (jnp.dot(aggr, wl_ref[...], preferred_element_type=jnp.float32)
         + jnp.dot(x_ref[...], wr_ref[...], preferred_element_type=jnp.float32)
         + b_ref[...])
    out_ref[...] = jnp.maximum(y, 0.0) if relu else y


def _tc_layer(summed, cnt, x, Wl, Wr, b, relu):
    grid = (N // BLK,)
    return pl.pallas_call(
        functools.partial(_tc_body, relu),
        grid=grid,
        in_specs=[
            pl.BlockSpec((NC, BLK, D), lambda i: (0, i, 0)),
            pl.BlockSpec((NC, BLK, 1), lambda i: (0, i, 0)),
            pl.BlockSpec((BLK, D), lambda i: (i, 0)),
            pl.BlockSpec((D, D), lambda i: (0, 0)),
            pl.BlockSpec((D, D), lambda i: (0, 0)),
            pl.BlockSpec((1, D), lambda i: (0, 0)),
        ],
        out_specs=pl.BlockSpec((BLK, D), lambda i: (i, 0)),
        out_shape=jax.ShapeDtypeStruct((N, D), jnp.float32),
    )(summed, cnt, x, Wl, Wr, b.reshape(1, D))


def kernel(x, edge_index, W1l, W1r, b1, W2l, W2r, b2):
    src = edge_index[0].astype(jnp.int32).reshape(NW, EPW)
    dst = edge_index[1].astype(jnp.int32).reshape(NW, NCH, C)
    (cnt2d,) = _sc_cnt(edge_index[1].astype(jnp.int32).reshape(NW, EPW),
                       jnp.zeros((NR, D), jnp.float32))
    cnt = cnt2d.reshape(NC, NR * D)[:, :N, None]
    (summed1,) = _sc_sum(x, src, dst)
    h = _tc_layer(summed1, cnt, x, W1l, W1r, b1, relu=True)
    (summed2,) = _sc_sum(h, src, dst)
    return _tc_layer(summed2, cnt, h, W2l, W2r, b2, relu=False)
